# baseline (speedup 1.0000x reference)
import jax
import jax.numpy as jnp
from jax import lax
from jax.experimental import pallas as pl
from jax.experimental.pallas import tpu as pltpu

N = 32768
DIM = 128
HEADS = 8
DH = 16
INNER = HEADS * DH
NSEG = 16
BLK = 4096
NB = N // BLK
EPS = 1e-6
BF = jnp.float32

_INTERPRET = False


def _dot_t(a, b):
    return lax.dot_general(a, b, (((1,), (1,)), ((), ())),
                           preferred_element_type=jnp.float32)


def _dot(a, b):
    return lax.dot_general(a, b, (((1,), (0,)), ((), ())),
                           preferred_element_type=jnp.float32)


def _fused_kernel(x_ref, b_ref, Wqkv_ref, A2_ref, bd_ref,
                  g_ref, bb_ref, Wout_ref, bout_ref,
                  y_ref, q_ref, ktv_ref, cnt_ref, m_ref, acc_ref):
    i = pl.program_id(0)
    bcol = b_ref[...]
    smin = jnp.min(bcol)
    smax = jnp.max(bcol)
    single = smin == smax
    cols = lax.broadcasted_iota(jnp.int32, (1, INNER), 1)

    @pl.when(i < NB)
    def _phase0():
        @pl.when(i == 0)
        def _init():
            ktv_ref[...] = jnp.zeros_like(ktv_ref)
            cnt_ref[...] = jnp.zeros_like(cnt_ref)

        x = x_ref[...]
        A2 = A2_ref[...]
        qkv = _dot_t(x, Wqkv_ref[...])
        q_ref[pl.ds(i * BLK, BLK), :] = qkv[:, :INNER].astype(BF)

        t = qkv[:, INNER:]
        m = _dot(t.astype(BF), A2)
        c = t - m
        var = _dot((c * c).astype(BF), A2)
        kvn = c * lax.rsqrt(var + EPS) * g_ref[...] + bb_ref[...]
        kt = kvn[:, :INNER].astype(BF).T
        vb = kvn[:, INNER:].astype(BF)

        E = (bcol == cols).astype(jnp.float32)
        cnt_ref[0:1, :] += jnp.sum(E, axis=0, keepdims=True)

        @pl.when(single)
        def _one_seg():
            sl = pl.ds(smin * INNER, INNER)
            ktv_ref[sl, :] += _dot(kt, vb)

        @pl.when(~single)
        def _multi_seg():
            Eb = E.astype(BF)
            for s in range(NSEG):
                @pl.when((s >= smin) & (s <= smax))
                def _acc(s=s):
                    ktv_ref[s * INNER:(s + 1) * INNER, :] += \
                        _dot(kt, vb * Eb[:, s:s + 1])

        @pl.when(i == NB - 1)
        def _finalize():
            inv = 1.0 / jnp.maximum(cnt_ref[0:1, :], 1.0)
            rows = lax.broadcasted_iota(jnp.int32, (NSEG * INNER, 1), 0)
            segcol = lax.broadcasted_iota(jnp.int32, (1, INNER), 1)
            S = ((rows // INNER) == segcol).astype(jnp.float32)
            P = _dot_t(S, inv)
            bd = bd_ref[...]
            Wout = Wout_ref[...]
            for s in range(NSEG):
                sl = slice(s * INNER, (s + 1) * INNER)
                ktv2 = ktv_ref[sl, :] * bd * P[sl, :]
                m_ref[sl, :] = _dot_t(ktv2.astype(BF), Wout).astype(BF)

    @pl.when(i >= NB)
    def _phase1():
        j = i - NB
        q = q_ref[pl.ds(j * BLK, BLK), :]
        bout = bout_ref[...]

        @pl.when(single)
        def _one_seg():
            sl = pl.ds(smin * INNER, INNER)
            y_ref[...] = _dot(q, m_ref[sl, :]) + bout

        @pl.when(~single)
        def _multi_seg():
            Eb = (bcol == cols).astype(BF)
            acc_ref[...] = jnp.zeros_like(acc_ref)
            for s in range(NSEG):
                @pl.when((s >= smin) & (s <= smax))
                def _acc(s=s):
                    acc_ref[...] += _dot(q * Eb[:, s:s + 1],
                                         m_ref[s * INNER:(s + 1) * INNER, :])
            y_ref[...] = acc_ref[...] + bout


def kernel(x, batch, Wqkv, g1, b1, g2, b2, Wout, bout):
    xf = x.reshape(N, DIM).astype(BF)
    bcol = batch.astype(jnp.int32).reshape(N, 1)
    A2 = jnp.kron(jnp.eye(2 * HEADS, dtype=jnp.float32),
                  jnp.ones((DH, DH), jnp.float32) / DH).astype(BF)
    bd = jnp.kron(jnp.eye(HEADS, dtype=jnp.float32),
                  jnp.ones((DH, DH), jnp.float32))
    gcat = jnp.concatenate([jnp.tile(g1, HEADS),
                            jnp.tile(g2, HEADS)]).reshape(1, 2 * INNER)
    bcat = jnp.concatenate([jnp.tile(b1, HEADS),
                            jnp.tile(b2, HEADS)]).reshape(1, 2 * INNER)
    bout_r = bout.reshape(1, DIM)

    def full(shape):
        return pl.BlockSpec(shape, lambda i: tuple(0 for _ in shape))

    rowblk_in = pl.BlockSpec(
        (BLK, DIM), lambda i: (jnp.where(i < NB, i, 0), 0))
    batblk = pl.BlockSpec((BLK, 1), lambda i: (lax.rem(i, NB), 0))
    rowblk_out = pl.BlockSpec(
        (BLK, DIM), lambda i: (jnp.where(i < NB, 0, i - NB), 0))

    y = pl.pallas_call(
        _fused_kernel,
        grid=(2 * NB,),
        in_specs=[rowblk_in, batblk, full((3 * INNER, DIM)),
                  full((2 * INNER, 2 * INNER)), full((DIM, DIM)),
                  full((1, 2 * INNER)), full((1, 2 * INNER)),
                  full((DIM, INNER)), full((1, DIM))],
        out_specs=rowblk_out,
        out_shape=jax.ShapeDtypeStruct((N, DIM), jnp.float32),
        scratch_shapes=[pltpu.VMEM((N, INNER), BF),
                        pltpu.VMEM((NSEG * INNER, INNER), jnp.float32),
                        pltpu.VMEM((8, INNER), jnp.float32),
                        pltpu.VMEM((NSEG * INNER, INNER), BF),
                        pltpu.VMEM((BLK, INNER), jnp.float32)],
        interpret=_INTERPRET,
    )(xf, bcol, Wqkv.astype(BF), A2, bd, gcat, bcat,
      Wout.astype(BF), bout_r)

    return y.reshape(1, N, DIM)


# speedup vs baseline: 1.0079x; 1.0079x over previous
import jax
import jax.numpy as jnp
from jax import lax
from jax.experimental import pallas as pl
from jax.experimental.pallas import tpu as pltpu

N = 32768
DIM = 128
HEADS = 8
DH = 16
INNER = HEADS * DH
NSEG = 16
BLK = 4096
NB = N // BLK
EPS = 1e-6
BF = jnp.float32

_INTERPRET = False


def _dot_t(a, b):
    return lax.dot_general(a, b, (((1,), (1,)), ((), ())),
                           preferred_element_type=jnp.float32)


def _dot(a, b):
    return lax.dot_general(a, b, (((1,), (0,)), ((), ())),
                           preferred_element_type=jnp.float32)


def _fused_kernel(x_ref, b_ref, Wqkv_ref, A2_ref,
                  g_ref, bb_ref, Wout_ref, bout_ref,
                  y_ref, q_ref, ktv_ref, cnt_ref, m_ref, acc_ref):
    i = pl.program_id(0)
    bcol = b_ref[...]
    smin = jnp.min(bcol)
    smax = jnp.max(bcol)
    single = smin == smax
    cols = lax.broadcasted_iota(jnp.int32, (1, INNER), 1)

    @pl.when(i < NB)
    def _phase0():
        @pl.when(i == 0)
        def _init():
            ktv_ref[...] = jnp.zeros_like(ktv_ref)
            cnt_ref[...] = jnp.zeros_like(cnt_ref)

        x = x_ref[...]
        A2 = A2_ref[...]
        qkv = _dot_t(x, Wqkv_ref[...])
        q_ref[pl.ds(i * BLK, BLK), :] = qkv[:, :INNER].astype(BF)

        t = qkv[:, INNER:]
        m = _dot(t.astype(BF), A2)
        c = t - m
        var = _dot((c * c).astype(BF), A2)
        kvn = c * lax.rsqrt(var + EPS) * g_ref[...] + bb_ref[...]
        kt = kvn[:, :INNER].astype(BF).T
        vb = kvn[:, INNER:].astype(BF)

        E = (bcol == cols).astype(jnp.float32)
        cnt_ref[0:1, :] += jnp.sum(E, axis=0, keepdims=True)

        @pl.when(single)
        def _one_seg():
            sl = pl.ds(smin * INNER, INNER)
            ktv_ref[sl, :] += _dot(kt, vb)

        @pl.when(~single)
        def _multi_seg():
            Eb = E.astype(BF)
            for s in range(NSEG):
                @pl.when((s >= smin) & (s <= smax))
                def _acc(s=s):
                    ktv_ref[s * INNER:(s + 1) * INNER, :] += \
                        _dot(kt, vb * Eb[:, s:s + 1])

        @pl.when(i == NB - 1)
        def _finalize():
            inv = 1.0 / jnp.maximum(cnt_ref[0:1, :], 1.0)
            rows = lax.broadcasted_iota(jnp.int32, (NSEG * INNER, 1), 0)
            segcol = lax.broadcasted_iota(jnp.int32, (1, INNER), 1)
            S = ((rows // INNER) == segcol).astype(jnp.float32)
            P = _dot_t(S, inv)
            bdt = (((rows // DH) % HEADS) == (segcol // DH)).astype(
                jnp.float32)
            ktv2 = ktv_ref[...] * bdt * P
            m_ref[...] = _dot_t(ktv2.astype(BF), Wout_ref[...]).astype(BF)

    @pl.when(i >= NB)
    def _phase1():
        j = i - NB
        q = q_ref[pl.ds(j * BLK, BLK), :]
        bout = bout_ref[...]

        @pl.when(single)
        def _one_seg():
            sl = pl.ds(smin * INNER, INNER)
            y_ref[...] = _dot(q, m_ref[sl, :]) + bout

        @pl.when(~single)
        def _multi_seg():
            Eb = (bcol == cols).astype(BF)
            acc_ref[...] = jnp.zeros_like(acc_ref)
            for s in range(NSEG):
                @pl.when((s >= smin) & (s <= smax))
                def _acc(s=s):
                    acc_ref[...] += _dot(q * Eb[:, s:s + 1],
                                         m_ref[s * INNER:(s + 1) * INNER, :])
            y_ref[...] = acc_ref[...] + bout


def kernel(x, batch, Wqkv, g1, b1, g2, b2, Wout, bout):
    xf = x.reshape(N, DIM).astype(BF)
    bcol = batch.astype(jnp.int32).reshape(N, 1)
    A2 = jnp.kron(jnp.eye(2 * HEADS, dtype=jnp.float32),
                  jnp.ones((DH, DH), jnp.float32) / DH).astype(BF)
    gcat = jnp.concatenate([jnp.tile(g1, HEADS),
                            jnp.tile(g2, HEADS)]).reshape(1, 2 * INNER)
    bcat = jnp.concatenate([jnp.tile(b1, HEADS),
                            jnp.tile(b2, HEADS)]).reshape(1, 2 * INNER)
    bout_r = bout.reshape(1, DIM)

    def full(shape):
        return pl.BlockSpec(shape, lambda i: tuple(0 for _ in shape))

    rowblk_in = pl.BlockSpec(
        (BLK, DIM), lambda i: (jnp.where(i < NB, i, 0), 0))
    batblk = pl.BlockSpec((BLK, 1), lambda i: (lax.rem(i, NB), 0))
    rowblk_out = pl.BlockSpec(
        (BLK, DIM), lambda i: (jnp.where(i < NB, 0, i - NB), 0))

    y = pl.pallas_call(
        _fused_kernel,
        grid=(2 * NB,),
        in_specs=[rowblk_in, batblk, full((3 * INNER, DIM)),
                  full((2 * INNER, 2 * INNER)),
                  full((1, 2 * INNER)), full((1, 2 * INNER)),
                  full((DIM, INNER)), full((1, DIM))],
        out_specs=rowblk_out,
        out_shape=jax.ShapeDtypeStruct((N, DIM), jnp.float32),
        scratch_shapes=[pltpu.VMEM((N, INNER), BF),
                        pltpu.VMEM((NSEG * INNER, INNER), jnp.float32),
                        pltpu.VMEM((8, INNER), jnp.float32),
                        pltpu.VMEM((NSEG * INNER, INNER), BF),
                        pltpu.VMEM((BLK, INNER), jnp.float32)],
        interpret=_INTERPRET,
    )(xf, bcol, Wqkv.astype(BF), A2, gcat, bcat,
      Wout.astype(BF), bout_r)

    return y.reshape(1, N, DIM)
